# baseline (speedup 1.0000x reference)
import functools

import jax
import jax.numpy as jnp
from jax import lax
from jax.experimental import pallas as pl
from jax.experimental.pallas import tpu as pltpu
from jax.experimental.pallas import tpu_sc as plsc

VOCAB = 1000000
EMB = 64
IMG = 512
OUT = 128
B = 4096
L = 200

NC = 2
NS = 16
NW = NC * NS
BPW = B // NW
HALF = L // 2
STEPS = 2 * BPW
NCH = EMB // 16

_mesh = plsc.VectorSubcoreMesh(core_axis_name="c", subcore_axis_name="s")


@functools.partial(
    pl.kernel,
    out_type=jax.ShapeDtypeStruct((B, EMB), jnp.float32),
    mesh=_mesh,
    scratch_types=[
        pltpu.VMEM((STEPS, HALF), jnp.int32),
        pltpu.VMEM((HALF, EMB), jnp.float32),
        pltpu.VMEM((HALF, EMB), jnp.float32),
        pltpu.VMEM((BPW, EMB), jnp.float32),
        pltpu.SemaphoreType.DMA,
        pltpu.SemaphoreType.DMA,
    ],
    compiler_params=pltpu.CompilerParams(use_tc_tiling_on_sc=False),
)
def _bow_sc(idx_hbm, table_hbm, out_hbm, idx_v, rows0, rows1, bow_v, sem0, sem1):
    wid = lax.axis_index("s") * NC + lax.axis_index("c")
    pltpu.sync_copy(idx_hbm.at[pl.ds(wid * STEPS, STEPS)], idx_v)
    pltpu.async_copy(table_hbm.at[idx_v.at[0]], rows0, sem0)
    pltpu.async_copy(table_hbm.at[idx_v.at[1]], rows1, sem1)

    def accum(rows):
        def row_body(r, accs):
            return tuple(a + rows[r, pl.ds(16 * c, 16)] for c, a in enumerate(accs))
        init = tuple(rows[0, pl.ds(16 * c, 16)] for c in range(NCH))
        return lax.fori_loop(1, HALF, row_body, init)

    def batch_body(i, carry):
        pltpu.make_async_copy(table_hbm.at[idx_v.at[0]], rows0, sem0).wait()
        accs = accum(rows0)
        for c in range(NCH):
            bow_v[i, pl.ds(16 * c, 16)] = accs[c]

        @pl.when(i < BPW - 1)
        def _():
            pltpu.async_copy(table_hbm.at[idx_v.at[2 * i + 2]], rows0, sem0)

        pltpu.make_async_copy(table_hbm.at[idx_v.at[1]], rows1, sem1).wait()
        accs = accum(rows1)
        for c in range(NCH):
            plsc.addupdate(bow_v.at[i, pl.ds(16 * c, 16)], accs[c])

        @pl.when(i < BPW - 1)
        def _():
            pltpu.async_copy(table_hbm.at[idx_v.at[2 * i + 3]], rows1, sem1)

        return carry

    lax.fori_loop(0, BPW, batch_body, None)
    pltpu.sync_copy(bow_v, out_hbm.at[pl.ds(wid * BPW, BPW)])


BLK = 512


def _dense_body(bow_ref, img_ref, wt_ref, b_ref, out_ref):
    logits = (
        jnp.dot(bow_ref[...], wt_ref[:EMB, :], preferred_element_type=jnp.float32)
        + jnp.dot(img_ref[...], wt_ref[EMB:, :], preferred_element_type=jnp.float32)
        + b_ref[...]
    )
    m = jnp.max(logits, axis=1, keepdims=True)
    x = logits - m
    out_ref[...] = x - jnp.log(jnp.sum(jnp.exp(x), axis=1, keepdims=True))


_dense_call = pl.pallas_call(
    _dense_body,
    grid=(B // BLK,),
    in_specs=[
        pl.BlockSpec((BLK, EMB), lambda i: (i, 0)),
        pl.BlockSpec((BLK, IMG), lambda i: (i, 0)),
        pl.BlockSpec((EMB + IMG, OUT), lambda i: (0, 0)),
        pl.BlockSpec((1, OUT), lambda i: (0, 0)),
    ],
    out_specs=pl.BlockSpec((BLK, OUT), lambda i: (i, 0)),
    out_shape=jax.ShapeDtypeStruct((B, OUT), jnp.float32),
)


@jax.jit
def kernel(word_features, image_features, emb_table, W, b):
    idx = word_features.astype(jnp.int32).reshape(NW * STEPS, HALF)
    bow = _bow_sc(idx, emb_table)
    return _dense_call(bow, image_features, W.T, b.reshape(1, OUT))


# speedup vs baseline: 1.6932x; 1.6932x over previous
import functools

import jax
import jax.numpy as jnp
from jax import lax
from jax.experimental import pallas as pl
from jax.experimental.pallas import tpu as pltpu
from jax.experimental.pallas import tpu_sc as plsc

VOCAB = 1000000
EMB = 64
IMG = 512
OUT = 128
B = 4096
L = 200

NC = 2
NS = 16
NW = NC * NS
BPW = B // NW
HALF = L // 2
STEPS = 2 * BPW
NCH = EMB // 16

TCHUNK = 8192
NCHUNKS = -(-(VOCAB + 1) // TCHUNK)
VPAD = NCHUNKS * TCHUNK

_mesh = plsc.VectorSubcoreMesh(core_axis_name="c", subcore_axis_name="s")


def _transpose_body(t_ref, out_ref):
    y = jnp.transpose(t_ref[...])
    out_ref[...] = jnp.concatenate([y[: TCHUNK // 2], y[TCHUNK // 2 :]], axis=1)


_transpose_call = pl.pallas_call(
    _transpose_body,
    grid=(NCHUNKS,),
    in_specs=[pl.BlockSpec((EMB, TCHUNK), lambda i: (0, i))],
    out_specs=pl.BlockSpec((TCHUNK // 2, 2 * EMB), lambda i: (i, 0)),
    out_shape=jax.ShapeDtypeStruct((VPAD // 2, 2 * EMB), jnp.float32),
)


@functools.partial(
    pl.kernel,
    out_type=jax.ShapeDtypeStruct((B, EMB), jnp.float32),
    mesh=_mesh,
    scratch_types=[
        pltpu.VMEM((STEPS, HALF), jnp.int32),
        pltpu.VMEM((HALF, EMB), jnp.float32),
        pltpu.VMEM((HALF, EMB), jnp.float32),
        pltpu.VMEM((BPW, EMB), jnp.float32),
        pltpu.SemaphoreType.DMA,
        pltpu.SemaphoreType.DMA,
    ],
    compiler_params=pltpu.CompilerParams(use_tc_tiling_on_sc=False),
)
def _bow_sc(idx_hbm, table_hbm, out_hbm, idx_v, rows0, rows1, bow_v, sem0, sem1):
    wid = lax.axis_index("s") * NC + lax.axis_index("c")
    pltpu.sync_copy(idx_hbm.at[pl.ds(wid * STEPS, STEPS)], idx_v)
    pltpu.async_copy(table_hbm.at[idx_v.at[0]], rows0, sem0)
    pltpu.async_copy(table_hbm.at[idx_v.at[1]], rows1, sem1)

    def accum(rows):
        def row_body(r, accs):
            return tuple(a + rows[r, pl.ds(16 * c, 16)] for c, a in enumerate(accs))
        init = tuple(rows[0, pl.ds(16 * c, 16)] for c in range(NCH))
        return lax.fori_loop(1, HALF, row_body, init)

    def batch_body(i, carry):
        pltpu.make_async_copy(table_hbm.at[idx_v.at[0]], rows0, sem0).wait()
        accs = accum(rows0)
        for c in range(NCH):
            bow_v[i, pl.ds(16 * c, 16)] = accs[c]

        @pl.when(i < BPW - 1)
        def _():
            pltpu.async_copy(table_hbm.at[idx_v.at[2 * i + 2]], rows0, sem0)

        pltpu.make_async_copy(table_hbm.at[idx_v.at[1]], rows1, sem1).wait()
        accs = accum(rows1)
        for c in range(NCH):
            plsc.addupdate(bow_v.at[i, pl.ds(16 * c, 16)], accs[c])

        @pl.when(i < BPW - 1)
        def _():
            pltpu.async_copy(table_hbm.at[idx_v.at[2 * i + 3]], rows1, sem1)

        return carry

    lax.fori_loop(0, BPW, batch_body, None)
    pltpu.sync_copy(bow_v, out_hbm.at[pl.ds(wid * BPW, BPW)])


BLK = 512


def _dense_body(bow_ref, img_ref, wt_ref, b_ref, out_ref):
    logits = (
        jnp.dot(bow_ref[...], wt_ref[:EMB, :], preferred_element_type=jnp.float32)
        + jnp.dot(img_ref[...], wt_ref[EMB:, :], preferred_element_type=jnp.float32)
        + b_ref[...]
    )
    m = jnp.max(logits, axis=1, keepdims=True)
    x = logits - m
    out_ref[...] = x - jnp.log(jnp.sum(jnp.exp(x), axis=1, keepdims=True))


_dense_call = pl.pallas_call(
    _dense_body,
    grid=(B // BLK,),
    in_specs=[
        pl.BlockSpec((BLK, EMB), lambda i: (i, 0)),
        pl.BlockSpec((BLK, IMG), lambda i: (i, 0)),
        pl.BlockSpec((EMB + IMG, OUT), lambda i: (0, 0)),
        pl.BlockSpec((1, OUT), lambda i: (0, 0)),
    ],
    out_specs=pl.BlockSpec((BLK, OUT), lambda i: (i, 0)),
    out_shape=jax.ShapeDtypeStruct((B, OUT), jnp.float32),
)


@jax.jit
def kernel(word_features, image_features, emb_table, W, b):
    v = word_features.astype(jnp.int32)
    c = v // TCHUNK
    j = v % TCHUNK
    row = c * TCHUNK + jnp.where(
        j < TCHUNK // 2, 2 * j, 2 * (j - TCHUNK // 2) + 1
    )
    idx = row.reshape(NW * STEPS, HALF)
    tlin = _transpose_call(emb_table.T)
    table = tlin.reshape(VPAD, EMB)
    bow = _bow_sc(idx, table)
    return _dense_call(bow, image_features, W.T, b.reshape(1, OUT))


# speedup vs baseline: 1.7377x; 1.0263x over previous
import functools

import jax
import jax.numpy as jnp
from jax import lax
from jax.experimental import pallas as pl
from jax.experimental.pallas import tpu as pltpu
from jax.experimental.pallas import tpu_sc as plsc

VOCAB = 1000000
EMB = 64
IMG = 512
OUT = 128
B = 4096
L = 200

NC = 2
NS = 16
NW = NC * NS
BPW = B // NW
HALF = L // 2
STEPS = 2 * BPW
NCH = EMB // 16

TCHUNK = 8192
NCHUNKS = -(-(VOCAB + 1) // TCHUNK)
VPAD = NCHUNKS * TCHUNK

_mesh = plsc.VectorSubcoreMesh(core_axis_name="c", subcore_axis_name="s")


def _transpose_body(t_ref, out_ref):
    y = jnp.transpose(t_ref[...])
    out_ref[...] = jnp.concatenate([y[: TCHUNK // 2], y[TCHUNK // 2 :]], axis=1)


_transpose_call = pl.pallas_call(
    _transpose_body,
    grid=(NCHUNKS,),
    in_specs=[pl.BlockSpec((EMB, TCHUNK), lambda i: (0, i))],
    out_specs=pl.BlockSpec((TCHUNK // 2, 2 * EMB), lambda i: (i, 0)),
    out_shape=jax.ShapeDtypeStruct((VPAD // 2, 2 * EMB), jnp.float32),
)


@functools.partial(
    pl.kernel,
    out_type=jax.ShapeDtypeStruct((B, EMB), jnp.float32),
    mesh=_mesh,
    scratch_types=[
        pltpu.VMEM((STEPS, HALF), jnp.int32),
        pltpu.VMEM((HALF, EMB), jnp.float32),
        pltpu.VMEM((HALF, EMB), jnp.float32),
        pltpu.VMEM((BPW, EMB), jnp.float32),
        pltpu.SemaphoreType.DMA,
        pltpu.SemaphoreType.DMA,
    ],
    compiler_params=pltpu.CompilerParams(use_tc_tiling_on_sc=False),
)
def _bow_sc(idx_hbm, table_hbm, out_hbm, idx_v, rows0, rows1, bow_v, sem0, sem1):
    wid = lax.axis_index("s") * NC + lax.axis_index("c")
    pltpu.sync_copy(idx_hbm.at[pl.ds(wid * STEPS, STEPS)], idx_v)
    pltpu.async_copy(table_hbm.at[idx_v.at[0]], rows0, sem0)
    pltpu.async_copy(table_hbm.at[idx_v.at[1]], rows1, sem1)

    def accum(rows):
        def add_row(r, accs):
            return tuple(a + rows[r, pl.ds(16 * c, 16)] for c, a in enumerate(accs))

        def row_body(g, accs):
            base = 4 * g
            for k in range(4):
                accs = add_row(base + k, accs)
            return accs

        accs = tuple(rows[0, pl.ds(16 * c, 16)] for c in range(NCH))
        for k in range(1, 4):
            accs = add_row(k, accs)
        return lax.fori_loop(1, HALF // 4, row_body, accs)

    def batch_body(i, carry):
        pltpu.make_async_copy(table_hbm.at[idx_v.at[0]], rows0, sem0).wait()
        accs = accum(rows0)
        for c in range(NCH):
            bow_v[i, pl.ds(16 * c, 16)] = accs[c]

        @pl.when(i < BPW - 1)
        def _():
            pltpu.async_copy(table_hbm.at[idx_v.at[2 * i + 2]], rows0, sem0)

        pltpu.make_async_copy(table_hbm.at[idx_v.at[1]], rows1, sem1).wait()
        accs = accum(rows1)
        for c in range(NCH):
            plsc.addupdate(bow_v.at[i, pl.ds(16 * c, 16)], accs[c])

        @pl.when(i < BPW - 1)
        def _():
            pltpu.async_copy(table_hbm.at[idx_v.at[2 * i + 3]], rows1, sem1)

        return carry

    lax.fori_loop(0, BPW, batch_body, None)
    pltpu.sync_copy(bow_v, out_hbm.at[pl.ds(wid * BPW, BPW)])


BLK = 512


def _dense_body(bow_ref, img_ref, wt_ref, b_ref, out_ref):
    logits = (
        jnp.dot(bow_ref[...], wt_ref[:EMB, :], preferred_element_type=jnp.float32)
        + jnp.dot(img_ref[...], wt_ref[EMB:, :], preferred_element_type=jnp.float32)
        + b_ref[...]
    )
    m = jnp.max(logits, axis=1, keepdims=True)
    x = logits - m
    out_ref[...] = x - jnp.log(jnp.sum(jnp.exp(x), axis=1, keepdims=True))


_dense_call = pl.pallas_call(
    _dense_body,
    grid=(B // BLK,),
    in_specs=[
        pl.BlockSpec((BLK, EMB), lambda i: (i, 0)),
        pl.BlockSpec((BLK, IMG), lambda i: (i, 0)),
        pl.BlockSpec((EMB + IMG, OUT), lambda i: (0, 0)),
        pl.BlockSpec((1, OUT), lambda i: (0, 0)),
    ],
    out_specs=pl.BlockSpec((BLK, OUT), lambda i: (i, 0)),
    out_shape=jax.ShapeDtypeStruct((B, OUT), jnp.float32),
)


@jax.jit
def kernel(word_features, image_features, emb_table, W, b):
    v = word_features.astype(jnp.int32)
    c = v // TCHUNK
    j = v % TCHUNK
    row = c * TCHUNK + jnp.where(
        j < TCHUNK // 2, 2 * j, 2 * (j - TCHUNK // 2) + 1
    )
    idx = row.reshape(NW * STEPS, HALF)
    tlin = _transpose_call(emb_table.T)
    table = tlin.reshape(VPAD, EMB)
    bow = _bow_sc(idx, table)
    return _dense_call(bow, image_features, W.T, b.reshape(1, OUT))


# speedup vs baseline: 1.8184x; 1.0464x over previous
import functools

import jax
import jax.numpy as jnp
from jax import lax
from jax.experimental import pallas as pl
from jax.experimental.pallas import tpu as pltpu
from jax.experimental.pallas import tpu_sc as plsc

VOCAB = 1000000
EMB = 64
IMG = 512
OUT = 128
B = 4096
L = 200

NC = 2
NS = 16
NW = NC * NS
BPW = B // NW
HALF = L // 2
STEPS = 2 * BPW
NCH = EMB // 16

TCHUNK = 16384
NCHUNKS = -(-(VOCAB + 1) // TCHUNK)
VPAD = NCHUNKS * TCHUNK

_mesh = plsc.VectorSubcoreMesh(core_axis_name="c", subcore_axis_name="s")


def _transpose_body(t_ref, out_ref):
    y = jnp.transpose(t_ref[...])
    out_ref[...] = jnp.concatenate([y[: TCHUNK // 2], y[TCHUNK // 2 :]], axis=1)


_transpose_call = pl.pallas_call(
    _transpose_body,
    grid=(NCHUNKS,),
    in_specs=[pl.BlockSpec((EMB, TCHUNK), lambda i: (0, i))],
    out_specs=pl.BlockSpec((TCHUNK // 2, 2 * EMB), lambda i: (i, 0)),
    out_shape=jax.ShapeDtypeStruct((VPAD // 2, 2 * EMB), jnp.float32),
)


@functools.partial(
    pl.kernel,
    out_type=jax.ShapeDtypeStruct((B, EMB), jnp.float32),
    mesh=_mesh,
    scratch_types=[
        pltpu.VMEM((STEPS, HALF), jnp.int32),
        pltpu.VMEM((HALF, EMB), jnp.float32),
        pltpu.VMEM((HALF, EMB), jnp.float32),
        pltpu.VMEM((BPW, EMB), jnp.float32),
        pltpu.SemaphoreType.DMA,
        pltpu.SemaphoreType.DMA,
    ],
    compiler_params=pltpu.CompilerParams(use_tc_tiling_on_sc=False),
)
def _bow_sc(idx_hbm, table_hbm, out_hbm, idx_v, rows0, rows1, bow_v, sem0, sem1):
    wid = lax.axis_index("s") * NC + lax.axis_index("c")
    pltpu.sync_copy(idx_hbm.at[pl.ds(wid * STEPS, STEPS)], idx_v)
    pltpu.async_copy(table_hbm.at[idx_v.at[0]], rows0, sem0)
    pltpu.async_copy(table_hbm.at[idx_v.at[1]], rows1, sem1)

    def accum(rows):
        zero = jnp.zeros((16,), jnp.float32)

        @plsc.parallel_loop(0, HALF, unroll=10, carry=(zero,) * NCH)
        def accs(r, acc):
            return tuple(a + rows[r, pl.ds(16 * c, 16)] for c, a in enumerate(acc))

        return accs

    def batch_body(i, carry):
        pltpu.make_async_copy(table_hbm.at[idx_v.at[0]], rows0, sem0).wait()
        accs = accum(rows0)
        for c in range(NCH):
            bow_v[i, pl.ds(16 * c, 16)] = accs[c]

        @pl.when(i < BPW - 1)
        def _():
            pltpu.async_copy(table_hbm.at[idx_v.at[2 * i + 2]], rows0, sem0)

        pltpu.make_async_copy(table_hbm.at[idx_v.at[1]], rows1, sem1).wait()
        accs = accum(rows1)
        for c in range(NCH):
            plsc.addupdate(bow_v.at[i, pl.ds(16 * c, 16)], accs[c])

        @pl.when(i < BPW - 1)
        def _():
            pltpu.async_copy(table_hbm.at[idx_v.at[2 * i + 3]], rows1, sem1)

        return carry

    lax.fori_loop(0, BPW, batch_body, None)
    pltpu.sync_copy(bow_v, out_hbm.at[pl.ds(wid * BPW, BPW)])


BLK = 512


def _dense_body(bow_ref, img_ref, wt_ref, b_ref, out_ref):
    logits = (
        jnp.dot(bow_ref[...], wt_ref[:EMB, :], preferred_element_type=jnp.float32)
        + jnp.dot(img_ref[...], wt_ref[EMB:, :], preferred_element_type=jnp.float32)
        + b_ref[...]
    )
    m = jnp.max(logits, axis=1, keepdims=True)
    x = logits - m
    out_ref[...] = x - jnp.log(jnp.sum(jnp.exp(x), axis=1, keepdims=True))


_dense_call = pl.pallas_call(
    _dense_body,
    grid=(B // BLK,),
    in_specs=[
        pl.BlockSpec((BLK, EMB), lambda i: (i, 0)),
        pl.BlockSpec((BLK, IMG), lambda i: (i, 0)),
        pl.BlockSpec((EMB + IMG, OUT), lambda i: (0, 0)),
        pl.BlockSpec((1, OUT), lambda i: (0, 0)),
    ],
    out_specs=pl.BlockSpec((BLK, OUT), lambda i: (i, 0)),
    out_shape=jax.ShapeDtypeStruct((B, OUT), jnp.float32),
)


@jax.jit
def kernel(word_features, image_features, emb_table, W, b):
    v = word_features.astype(jnp.int32)
    c = v // TCHUNK
    j = v % TCHUNK
    row = c * TCHUNK + jnp.where(
        j < TCHUNK // 2, 2 * j, 2 * (j - TCHUNK // 2) + 1
    )
    idx = row.reshape(NW * STEPS, HALF)
    tlin = _transpose_call(emb_table.T)
    table = tlin.reshape(VPAD, EMB)
    bow = _bow_sc(idx, table)
    return _dense_call(bow, image_features, W.T, b.reshape(1, OUT))


# speedup vs baseline: 1.9931x; 1.0961x over previous
import functools

import jax
import jax.numpy as jnp
from jax import lax
from jax.experimental import pallas as pl
from jax.experimental.pallas import tpu as pltpu
from jax.experimental.pallas import tpu_sc as plsc

VOCAB = 1000000
EMB = 64
IMG = 512
OUT = 128
B = 4096
L = 200

NC = 2
NS = 16
NW = NC * NS
BPW = B // NW
HALF = L // 2
STEPS = 2 * BPW
NCH = EMB // 16

TCHUNK = 16384
NCHUNKS = -(-(VOCAB + 1) // TCHUNK)
VPAD = NCHUNKS * TCHUNK

_mesh = plsc.VectorSubcoreMesh(core_axis_name="c", subcore_axis_name="s")


def _transpose_body(t_ref, out_ref):
    y = jnp.transpose(t_ref[...])
    out_ref[...] = jnp.concatenate([y[: TCHUNK // 2], y[TCHUNK // 2 :]], axis=1)


_transpose_call = pl.pallas_call(
    _transpose_body,
    grid=(NCHUNKS,),
    in_specs=[pl.BlockSpec((EMB, TCHUNK), lambda i: (0, i))],
    out_specs=pl.BlockSpec((TCHUNK // 2, 2 * EMB), lambda i: (i, 0)),
    out_shape=jax.ShapeDtypeStruct((VPAD // 2, 2 * EMB), jnp.float32),
)


@functools.partial(
    pl.kernel,
    out_type=jax.ShapeDtypeStruct((B, EMB), jnp.float32),
    mesh=_mesh,
    scratch_types=[
        pltpu.VMEM((STEPS, HALF), jnp.int32),
        pltpu.VMEM((L, EMB), jnp.float32),
        pltpu.VMEM((L, EMB), jnp.float32),
        pltpu.VMEM((BPW, EMB), jnp.float32),
        pltpu.SemaphoreType.DMA,
        pltpu.SemaphoreType.DMA,
    ],
    compiler_params=pltpu.CompilerParams(use_tc_tiling_on_sc=False),
)
def _bow_sc(idx_hbm, table_hbm, out_hbm, idx_v, rows_a, rows_b, bow_v, sem_a, sem_b):
    wid = lax.axis_index("s") * NC + lax.axis_index("c")
    pltpu.sync_copy(idx_hbm.at[pl.ds(wid * STEPS, STEPS)], idx_v)

    def issue(i, rows, sem):
        pltpu.async_copy(table_hbm.at[idx_v.at[2 * i]], rows.at[pl.ds(0, HALF)], sem)
        pltpu.async_copy(
            table_hbm.at[idx_v.at[2 * i + 1]], rows.at[pl.ds(HALF, HALF)], sem
        )

    def wait(rows, sem):
        pltpu.make_async_copy(
            table_hbm.at[idx_v.at[0]], rows.at[pl.ds(0, HALF)], sem
        ).wait()
        pltpu.make_async_copy(
            table_hbm.at[idx_v.at[0]], rows.at[pl.ds(HALF, HALF)], sem
        ).wait()

    issue(0, rows_a, sem_a)
    issue(1, rows_b, sem_b)

    def accum(rows):
        zero = jnp.zeros((16,), jnp.float32)

        @plsc.parallel_loop(0, L, unroll=10, carry=(zero,) * NCH)
        def accs(r, acc):
            return tuple(a + rows[r, pl.ds(16 * c, 16)] for c, a in enumerate(acc))

        return accs

    def pair_body(g, carry):
        for p, (rows, sem) in enumerate(((rows_a, sem_a), (rows_b, sem_b))):
            i = 2 * g + p
            wait(rows, sem)
            accs = accum(rows)
            for c in range(NCH):
                bow_v[i, pl.ds(16 * c, 16)] = accs[c]

            @pl.when(i < BPW - 2)
            def _():
                issue(i + 2, rows, sem)

        return carry

    lax.fori_loop(0, BPW // 2, pair_body, None)
    pltpu.sync_copy(bow_v, out_hbm.at[pl.ds(wid * BPW, BPW)])


BLK = 512


def _dense_body(bow_ref, img_ref, wt_ref, b_ref, out_ref):
    logits = (
        jnp.dot(bow_ref[...], wt_ref[:EMB, :], preferred_element_type=jnp.float32)
        + jnp.dot(img_ref[...], wt_ref[EMB:, :], preferred_element_type=jnp.float32)
        + b_ref[...]
    )
    m = jnp.max(logits, axis=1, keepdims=True)
    x = logits - m
    out_ref[...] = x - jnp.log(jnp.sum(jnp.exp(x), axis=1, keepdims=True))


_dense_call = pl.pallas_call(
    _dense_body,
    grid=(B // BLK,),
    in_specs=[
        pl.BlockSpec((BLK, EMB), lambda i: (i, 0)),
        pl.BlockSpec((BLK, IMG), lambda i: (i, 0)),
        pl.BlockSpec((EMB + IMG, OUT), lambda i: (0, 0)),
        pl.BlockSpec((1, OUT), lambda i: (0, 0)),
    ],
    out_specs=pl.BlockSpec((BLK, OUT), lambda i: (i, 0)),
    out_shape=jax.ShapeDtypeStruct((B, OUT), jnp.float32),
)


@jax.jit
def kernel(word_features, image_features, emb_table, W, b):
    v = word_features.astype(jnp.int32)
    c = v // TCHUNK
    j = v % TCHUNK
    row = c * TCHUNK + jnp.where(
        j < TCHUNK // 2, 2 * j, 2 * (j - TCHUNK // 2) + 1
    )
    idx = row.reshape(NW * STEPS, HALF)
    tlin = _transpose_call(emb_table.T)
    table = tlin.reshape(VPAD, EMB)
    bow = _bow_sc(idx, table)
    return _dense_call(bow, image_features, W.T, b.reshape(1, OUT))


# speedup vs baseline: 2.0754x; 1.0413x over previous
import functools

import jax
import jax.numpy as jnp
from jax import lax
from jax.experimental import pallas as pl
from jax.experimental.pallas import tpu as pltpu
from jax.experimental.pallas import tpu_sc as plsc

VOCAB = 1000000
EMB = 64
IMG = 512
OUT = 128
B = 4096
L = 200

NC = 2
NS = 16
NW = NC * NS
BPW = B // NW
HALF = L // 2
STEPS = 2 * BPW
NCH = EMB // 16

TCHUNK = 32768
NCHUNKS = -(-(VOCAB + 1) // TCHUNK)
VPAD = NCHUNKS * TCHUNK

_mesh = plsc.VectorSubcoreMesh(core_axis_name="c", subcore_axis_name="s")


def _transpose_body(t_ref, out_ref):
    y = jnp.transpose(t_ref[...])
    out_ref[...] = jnp.concatenate([y[: TCHUNK // 2], y[TCHUNK // 2 :]], axis=1)


_transpose_call = pl.pallas_call(
    _transpose_body,
    grid=(NCHUNKS,),
    in_specs=[pl.BlockSpec((EMB, TCHUNK), lambda i: (0, i))],
    out_specs=pl.BlockSpec((TCHUNK // 2, 2 * EMB), lambda i: (i, 0)),
    out_shape=jax.ShapeDtypeStruct((VPAD // 2, 2 * EMB), jnp.float32),
)


@functools.partial(
    pl.kernel,
    out_type=jax.ShapeDtypeStruct((B, EMB), jnp.float32),
    mesh=_mesh,
    scratch_types=[
        pltpu.VMEM((STEPS, HALF), jnp.int32),
        pltpu.VMEM((L, EMB), jnp.float32),
        pltpu.VMEM((L, EMB), jnp.float32),
        pltpu.VMEM((BPW, EMB), jnp.float32),
        pltpu.SemaphoreType.DMA,
        pltpu.SemaphoreType.DMA,
    ],
    compiler_params=pltpu.CompilerParams(use_tc_tiling_on_sc=False),
)
def _bow_sc(idx_hbm, table_hbm, out_hbm, idx_v, rows_a, rows_b, bow_v, sem_a, sem_b):
    wid = lax.axis_index("s") * NC + lax.axis_index("c")
    pltpu.sync_copy(idx_hbm.at[pl.ds(wid * STEPS, STEPS)], idx_v)

    def issue(i, rows, sem):
        pltpu.async_copy(table_hbm.at[idx_v.at[2 * i]], rows.at[pl.ds(0, HALF)], sem)
        pltpu.async_copy(
            table_hbm.at[idx_v.at[2 * i + 1]], rows.at[pl.ds(HALF, HALF)], sem
        )

    def wait(rows, sem):
        pltpu.make_async_copy(
            table_hbm.at[idx_v.at[0]], rows.at[pl.ds(0, HALF)], sem
        ).wait()
        pltpu.make_async_copy(
            table_hbm.at[idx_v.at[0]], rows.at[pl.ds(HALF, HALF)], sem
        ).wait()

    issue(0, rows_a, sem_a)
    issue(1, rows_b, sem_b)

    def accum(rows):
        zero = jnp.zeros((16,), jnp.float32)

        @plsc.parallel_loop(0, L, unroll=20, carry=(zero,) * NCH)
        def accs(r, acc):
            return tuple(a + rows[r, pl.ds(16 * c, 16)] for c, a in enumerate(acc))

        return accs

    def pair_body(g, carry):
        for p, (rows, sem) in enumerate(((rows_a, sem_a), (rows_b, sem_b))):
            i = 2 * g + p
            wait(rows, sem)
            accs = accum(rows)
            for c in range(NCH):
                bow_v[i, pl.ds(16 * c, 16)] = accs[c]

            @pl.when(i < BPW - 2)
            def _():
                issue(i + 2, rows, sem)

        return carry

    lax.fori_loop(0, BPW // 2, pair_body, None)
    pltpu.sync_copy(bow_v, out_hbm.at[pl.ds(wid * BPW, BPW)])


BLK = 512


def _dense_body(bow_ref, img_ref, wt_ref, b_ref, out_ref):
    logits = (
        jnp.dot(bow_ref[...], wt_ref[:EMB, :], preferred_element_type=jnp.float32)
        + jnp.dot(img_ref[...], wt_ref[EMB:, :], preferred_element_type=jnp.float32)
        + b_ref[...]
    )
    m = jnp.max(logits, axis=1, keepdims=True)
    x = logits - m
    out_ref[...] = x - jnp.log(jnp.sum(jnp.exp(x), axis=1, keepdims=True))


_dense_call = pl.pallas_call(
    _dense_body,
    grid=(B // BLK,),
    in_specs=[
        pl.BlockSpec((BLK, EMB), lambda i: (i, 0)),
        pl.BlockSpec((BLK, IMG), lambda i: (i, 0)),
        pl.BlockSpec((EMB + IMG, OUT), lambda i: (0, 0)),
        pl.BlockSpec((1, OUT), lambda i: (0, 0)),
    ],
    out_specs=pl.BlockSpec((BLK, OUT), lambda i: (i, 0)),
    out_shape=jax.ShapeDtypeStruct((B, OUT), jnp.float32),
)


@jax.jit
def kernel(word_features, image_features, emb_table, W, b):
    v = word_features.astype(jnp.int32)
    c = v // TCHUNK
    j = v % TCHUNK
    row = c * TCHUNK + jnp.where(
        j < TCHUNK // 2, 2 * j, 2 * (j - TCHUNK // 2) + 1
    )
    idx = row.reshape(NW * STEPS, HALF)
    tlin = _transpose_call(emb_table.T)
    table = tlin.reshape(VPAD, EMB)
    bow = _bow_sc(idx, table)
    return _dense_call(bow, image_features, W.T, b.reshape(1, OUT))


# speedup vs baseline: 2.2436x; 1.0810x over previous
import functools

import jax
import jax.numpy as jnp
from jax import lax
from jax.experimental import pallas as pl
from jax.experimental.pallas import tpu as pltpu
from jax.experimental.pallas import tpu_sc as plsc

VOCAB = 1000000
EMB = 64
IMG = 512
OUT = 128
B = 4096
L = 200

NC = 2
NS = 16
NW = NC * NS
BPW = B // NW
HALF = L // 2
STEPS = 2 * BPW
NCH = EMB // 16

TCHUNK = 32768
NCHUNKS = -(-(VOCAB + 1) // TCHUNK)
VPAD = NCHUNKS * TCHUNK

_mesh = plsc.VectorSubcoreMesh(core_axis_name="c", subcore_axis_name="s")


def _transpose_body(t_ref, out_ref):
    y = jnp.transpose(t_ref[...])
    out_ref[...] = jnp.concatenate([y[: TCHUNK // 2], y[TCHUNK // 2 :]], axis=1)


_transpose_call = pl.pallas_call(
    _transpose_body,
    grid=(NCHUNKS,),
    in_specs=[pl.BlockSpec((EMB, TCHUNK), lambda i: (0, i))],
    out_specs=pl.BlockSpec((TCHUNK // 2, 2 * EMB), lambda i: (i, 0)),
    out_shape=jax.ShapeDtypeStruct((VPAD // 2, 2 * EMB), jnp.float32),
)


@functools.partial(
    pl.kernel,
    out_type=jax.ShapeDtypeStruct((B, EMB), jnp.float32),
    mesh=_mesh,
    scratch_types=[
        pltpu.VMEM((STEPS, HALF), jnp.int32),
        pltpu.VMEM((4, L, EMB), jnp.float32),
        pltpu.VMEM((BPW, EMB), jnp.float32),
        pltpu.SemaphoreType.DMA,
        pltpu.SemaphoreType.DMA,
        pltpu.SemaphoreType.DMA,
        pltpu.SemaphoreType.DMA,
    ],
    compiler_params=pltpu.CompilerParams(use_tc_tiling_on_sc=False),
)
def _bow_sc(idx_hbm, table_hbm, out_hbm, idx_v, ring, bow_v, s0, s1, s2, s3):
    wid = lax.axis_index("s") * NC + lax.axis_index("c")
    sems = (s0, s1, s2, s3)
    pltpu.sync_copy(idx_hbm.at[pl.ds(wid * STEPS, STEPS)], idx_v)

    def issue(i, p, sem):
        pltpu.async_copy(
            table_hbm.at[idx_v.at[2 * i]], ring.at[p, pl.ds(0, HALF)], sem
        )
        pltpu.async_copy(
            table_hbm.at[idx_v.at[2 * i + 1]], ring.at[p, pl.ds(HALF, HALF)], sem
        )

    def wait(p, sem):
        pltpu.make_async_copy(
            table_hbm.at[idx_v.at[0]], ring.at[p, pl.ds(0, HALF)], sem
        ).wait()
        pltpu.make_async_copy(
            table_hbm.at[idx_v.at[0]], ring.at[p, pl.ds(HALF, HALF)], sem
        ).wait()

    for p in range(4):
        issue(p, p, sems[p])

    def accum(rows):
        zero = jnp.zeros((16,), jnp.float32)

        @plsc.parallel_loop(0, L, unroll=20, carry=(zero,) * NCH)
        def accs(r, acc):
            return tuple(a + rows[r, pl.ds(16 * c, 16)] for c, a in enumerate(acc))

        return accs

    def quad_body(g, carry):
        for p in range(4):
            i = 4 * g + p
            wait(p, sems[p])
            accs = accum(ring.at[p])
            for c in range(NCH):
                bow_v[i, pl.ds(16 * c, 16)] = accs[c]

            @pl.when(i < BPW - 4)
            def _():
                issue(i + 4, p, sems[p])

        return carry

    lax.fori_loop(0, BPW // 4, quad_body, None)
    pltpu.sync_copy(bow_v, out_hbm.at[pl.ds(wid * BPW, BPW)])


BLK = 512


def _dense_body(bow_ref, img_ref, wt_ref, b_ref, out_ref):
    logits = (
        jnp.dot(bow_ref[...], wt_ref[:EMB, :], preferred_element_type=jnp.float32)
        + jnp.dot(img_ref[...], wt_ref[EMB:, :], preferred_element_type=jnp.float32)
        + b_ref[...]
    )
    m = jnp.max(logits, axis=1, keepdims=True)
    x = logits - m
    out_ref[...] = x - jnp.log(jnp.sum(jnp.exp(x), axis=1, keepdims=True))


_dense_call = pl.pallas_call(
    _dense_body,
    grid=(B // BLK,),
    in_specs=[
        pl.BlockSpec((BLK, EMB), lambda i: (i, 0)),
        pl.BlockSpec((BLK, IMG), lambda i: (i, 0)),
        pl.BlockSpec((EMB + IMG, OUT), lambda i: (0, 0)),
        pl.BlockSpec((1, OUT), lambda i: (0, 0)),
    ],
    out_specs=pl.BlockSpec((BLK, OUT), lambda i: (i, 0)),
    out_shape=jax.ShapeDtypeStruct((B, OUT), jnp.float32),
)


@jax.jit
def kernel(word_features, image_features, emb_table, W, b):
    v = word_features.astype(jnp.int32)
    c = v // TCHUNK
    j = v % TCHUNK
    row = c * TCHUNK + jnp.where(
        j < TCHUNK // 2, 2 * j, 2 * (j - TCHUNK // 2) + 1
    )
    idx = row.reshape(NW * STEPS, HALF)
    tlin = _transpose_call(emb_table.T)
    table = tlin.reshape(VPAD, EMB)
    bow = _bow_sc(idx, table)
    return _dense_call(bow, image_features, W.T, b.reshape(1, OUT))
